# baseline (speedup 1.0000x reference)
import functools

import jax
import jax.numpy as jnp
from jax import lax
from jax.experimental import pallas as pl
from jax.experimental.pallas import tpu as pltpu
from jax.experimental.pallas import tpu_sc as plsc

B = 16
T = 1024
K = 4096
D = 64
HOP = 128
C = B * T
F = 4 * C

NC = 2
NS = 16
NW = NC * NS
C_W = C // NW
F_W = 4 * C_W
NSUB = 8
C_SUB = C_W // NSUB
F_SUB = 4 * C_SUB
SEG = 128
NSEG = F_SUB // SEG
NSEG_W = F_W // SEG


def _sc_gather_combine(ids_flat, cb1, cb2, cb3):
    mesh = plsc.VectorSubcoreMesh(core_axis_name="c", subcore_axis_name="s")

    H_SUB = F_SUB // 2
    NSEG2 = F_W // 2 // SEG
    NSEG2_SUB = NSEG2 // NSUB or 1

    @functools.partial(
        pl.kernel,
        out_type=jax.ShapeDtypeStruct((F, 2 * D), jnp.bfloat16),
        mesh=mesh,
        scratch_types=[
            pltpu.VMEM((C_W * 7,), jnp.int32),
            pltpu.VMEM((NSEG_W, SEG), jnp.int32),
            pltpu.VMEM((NSEG2, SEG), jnp.int32),
            pltpu.VMEM((NSUB, C_SUB), jnp.int32),
            pltpu.VMEM((2, F_SUB, D), jnp.bfloat16),
            pltpu.VMEM((2, H_SUB, D), jnp.bfloat16),
            pltpu.VMEM((2, C_SUB, D), jnp.bfloat16),
            pltpu.SemaphoreType.DMA,
            pltpu.SemaphoreType.DMA,
            pltpu.SemaphoreType.DMA,
            pltpu.SemaphoreType.DMA,
            pltpu.SemaphoreType.DMA,
        ],
        compiler_params=pltpu.CompilerParams(needs_layout_passes=False,
                                             use_tc_tiling_on_sc=False),
    )
    def k(ids_hbm, cb1_hbm, cb2_hbm, cb3_hbm, z_hbm,
          ids_v, idx3_v, idx2_v, idx1_v, z_v, l2_v, l1_v,
          sem_ids, sem_g0, sem_g1, sem_aux, sem_out):
        wid = lax.axis_index("s") * NC + lax.axis_index("c")
        pltpu.async_copy(ids_hbm.at[pl.ds(wid * C_W * 7, C_W * 7)],
                         ids_v, sem_ids).wait()
        lane = lax.broadcasted_iota(jnp.int32, (16,), 0)

        def build3(i, carry):
            f = lane + i * 16
            s = f & 3
            idx3_v[i >> 3, pl.ds((i & 7) * 16, 16)] = plsc.load_gather(
                ids_v, [(f >> 2) * 7 + (3 + s)]) - 2 * K
            return carry

        def build2(i, carry):
            h = lane + i * 16
            idx2_v[i >> 3, pl.ds((i & 7) * 16, 16)] = plsc.load_gather(
                ids_v, [(h >> 1) * 7 + (1 + (h & 1))]) - K
            return carry

        def build1(i, carry):
            t = lane + i * 16
            idx1_v[i >> 2, pl.ds((i & 3) * 16, 16)] = plsc.load_gather(
                ids_v, [t * 7])
            return carry

        lax.fori_loop(0, F_W // 16, build3, 0)
        lax.fori_loop(0, F_W // 2 // 16, build2, 0)
        lax.fori_loop(0, F_W // 4 // 16, build1, 0)

        sem_g = (sem_g0, sem_g1)

        def fire(sub, buf):
            ds_ = [pltpu.async_copy(
                       cb3_hbm.at[idx3_v.at[NSEG * sub + g]],
                       z_v.at[buf, pl.ds(g * SEG, SEG)],
                       sem_g[buf])
                   for g in range(NSEG)]
            ds_ += [pltpu.async_copy(
                        cb2_hbm.at[idx2_v.at[NSEG2_SUB * sub + g]],
                        l2_v.at[buf, pl.ds(g * SEG, SEG)],
                        sem_aux)
                    for g in range(NSEG2_SUB)]
            ds_.append(pltpu.async_copy(
                cb1_hbm.at[idx1_v.at[sub]], l1_v.at[buf], sem_aux))
            return ds_

        def fire_out(sub, buf):
            base_f = wid * F_W + sub * F_SUB
            return pltpu.async_copy(
                z_v.at[buf],
                z_hbm.at[pl.ds(base_f, F_SUB), pl.ds(0, D)], sem_out)

        def add_phase(buf):
            def body(tc, carry):
                a = [l1_v[buf, tc, pl.ds(c * 32, 32)] for c in range(2)]
                for u in range(2):
                    acc = [a[c] + l2_v[buf, 2 * tc + u, pl.ds(c * 32, 32)]
                           for c in range(2)]
                    for s2 in range(2):
                        fr = 4 * tc + 2 * u + s2
                        for c in range(2):
                            sl = pl.ds(c * 32, 32)
                            z_v[buf, fr, sl] = z_v[buf, fr, sl] + acc[c]
                return carry
            lax.fori_loop(0, C_SUB, body, 0)

        gat_d = [None] * NSUB
        out_d = [None] * NSUB
        gat_d[0] = fire(0, 0)
        for sub in range(NSUB):
            buf = sub % 2
            for dsc in gat_d[sub]:
                dsc.wait()
            if sub + 1 < NSUB:
                if sub >= 1:
                    out_d[sub - 1].wait()
                gat_d[sub + 1] = fire(sub + 1, 1 - buf)
            add_phase(buf)
            out_d[sub] = fire_out(sub, buf)
        out_d[NSUB - 2].wait()
        out_d[NSUB - 1].wait()

    return k(ids_flat, cb1, cb2, cb3)


def _tc_decode(z, W_dec, b_dec):
    ROWS = 16384

    def body(z_ref, w_ref, b_ref, o_ref):
        x = z_ref[:, :D].astype(jnp.float32)
        acc = jnp.dot(x, w_ref[...], preferred_element_type=jnp.float32)
        o_ref[...] = jnp.tanh(acc + b_ref[...])

    return pl.pallas_call(
        body,
        grid=(F // ROWS,),
        in_specs=[
            pl.BlockSpec((ROWS, 2 * D), lambda i: (i, 0)),
            pl.BlockSpec((D, HOP), lambda i: (0, 0)),
            pl.BlockSpec((1, HOP), lambda i: (0, 0)),
        ],
        out_specs=pl.BlockSpec((ROWS, HOP), lambda i: (i, 0)),
        out_shape=jax.ShapeDtypeStruct((F, HOP), jnp.float32),
    )(z, W_dec, b_dec.reshape(1, HOP))


def kernel(ids, cb1, cb2, cb3, W_dec, b_dec):
    ids_flat = ids.reshape(-1).astype(jnp.int32)
    z = _sc_gather_combine(ids_flat,
                           cb1.astype(jnp.bfloat16),
                           cb2.astype(jnp.bfloat16),
                           cb3.astype(jnp.bfloat16))
    out = _tc_decode(z, W_dec, b_dec)
    return out.reshape(B, 1, 4 * T * HOP)


# speedup vs baseline: 1.8155x; 1.8155x over previous
import functools

import jax
import jax.numpy as jnp
from jax import lax
from jax.experimental import pallas as pl
from jax.experimental.pallas import tpu as pltpu
from jax.experimental.pallas import tpu_sc as plsc

B = 16
T = 1024
K = 4096
D = 64
HOP = 128
C = B * T
F = 4 * C
CH = 2

NC = 2
NS = 16
NW = NC * NS
C_CH = C // CH
F_CH = 4 * C_CH
C_W = C_CH // NW
F_W = 4 * C_W
NSUB = 4
C_SUB = C_W // NSUB
F_SUB = 4 * C_SUB
H_SUB = F_SUB // 2
SEG = 128
NSEG = F_SUB // SEG
NSEG_W = F_W // SEG
NSEG2 = F_W // 2 // SEG


def _sc_gather_combine(ids_flat, cb1, cb2, cb3, h):
    mesh = plsc.VectorSubcoreMesh(core_axis_name="c", subcore_axis_name="s")

    @functools.partial(
        pl.kernel,
        out_type=jax.ShapeDtypeStruct((F_CH, 2 * D), jnp.float32),
        mesh=mesh,
        scratch_types=[
            pltpu.VMEM((C_W * 7,), jnp.int32),
            pltpu.VMEM((NSEG_W, SEG), jnp.int32),
            pltpu.VMEM((NSEG2, SEG), jnp.int32),
            pltpu.VMEM((NSUB, C_SUB), jnp.int32),
            pltpu.VMEM((2, F_SUB, D), jnp.float32),
            pltpu.VMEM((2, H_SUB, D), jnp.float32),
            pltpu.VMEM((2, C_SUB, D), jnp.float32),
            pltpu.SemaphoreType.DMA,
            pltpu.SemaphoreType.DMA,
            pltpu.SemaphoreType.DMA,
            pltpu.SemaphoreType.DMA,
            pltpu.SemaphoreType.DMA,
        ],
        compiler_params=pltpu.CompilerParams(needs_layout_passes=False,
                                             use_tc_tiling_on_sc=False),
    )
    def k(ids_hbm, cb1_hbm, cb2_hbm, cb3_hbm, z_hbm,
          ids_v, idx3_v, idx2_v, idx1_v, z_v, l2_v, l1_v,
          sem_ids, sem_g0, sem_g1, sem_aux, sem_out):
        wid = lax.axis_index("s") * NC + lax.axis_index("c")
        base_c = h * C_CH + wid * C_W
        pltpu.async_copy(ids_hbm.at[pl.ds(base_c * 7, C_W * 7)],
                         ids_v, sem_ids).wait()
        lane = lax.broadcasted_iota(jnp.int32, (16,), 0)

        def build3(i, carry):
            f = lane + i * 16
            s = f & 3
            idx3_v[i >> 3, pl.ds((i & 7) * 16, 16)] = plsc.load_gather(
                ids_v, [(f >> 2) * 7 + (3 + s)]) - 2 * K
            return carry

        def build2(i, carry):
            hh = lane + i * 16
            idx2_v[i >> 3, pl.ds((i & 7) * 16, 16)] = plsc.load_gather(
                ids_v, [(hh >> 1) * 7 + (1 + (hh & 1))]) - K
            return carry

        def build1(i, carry):
            t = lane + i * 16
            idx1_v[i >> 2, pl.ds((i & 3) * 16, 16)] = plsc.load_gather(
                ids_v, [t * 7])
            return carry

        lax.fori_loop(0, F_W // 16, build3, 0)
        lax.fori_loop(0, F_W // 2 // 16, build2, 0)
        lax.fori_loop(0, F_W // 4 // 16, build1, 0)

        sem_g = (sem_g0, sem_g1)

        def fire(sub, buf):
            ds_ = [pltpu.async_copy(
                       cb3_hbm.at[idx3_v.at[NSEG * sub + g]],
                       z_v.at[buf, pl.ds(g * SEG, SEG)],
                       sem_g[buf])
                   for g in range(NSEG)]
            ds_.append(pltpu.async_copy(
                cb2_hbm.at[idx2_v.at[sub]], l2_v.at[buf], sem_aux))
            ds_.append(pltpu.async_copy(
                cb1_hbm.at[idx1_v.at[sub]], l1_v.at[buf], sem_aux))
            return ds_

        def fire_out(sub, buf):
            base_f = wid * F_W + sub * F_SUB
            return pltpu.async_copy(
                z_v.at[buf],
                z_hbm.at[pl.ds(base_f, F_SUB), pl.ds(0, D)], sem_out)

        def add_phase(buf):
            def body(tc, carry):
                a = [l1_v[buf, tc, pl.ds(c * 16, 16)] for c in range(4)]
                for u in range(2):
                    acc = [a[c] + l2_v[buf, 2 * tc + u, pl.ds(c * 16, 16)]
                           for c in range(4)]
                    for s2 in range(2):
                        fr = 4 * tc + 2 * u + s2
                        for c in range(4):
                            plsc.addupdate(
                                z_v.at[buf, fr, pl.ds(c * 16, 16)], acc[c])
                return carry
            lax.fori_loop(0, C_SUB, body, 0)

        gat_d = [None] * NSUB
        out_d = [None] * NSUB
        gat_d[0] = fire(0, 0)
        for sub in range(NSUB):
            buf = sub % 2
            for dsc in gat_d[sub]:
                dsc.wait()
            if sub + 1 < NSUB:
                if sub >= 1:
                    out_d[sub - 1].wait()
                gat_d[sub + 1] = fire(sub + 1, 1 - buf)
            add_phase(buf)
            out_d[sub] = fire_out(sub, buf)
        out_d[NSUB - 2].wait()
        out_d[NSUB - 1].wait()

    return k(ids_flat, cb1, cb2, cb3)


ROWS = 8192
STEPS = F_CH // ROWS


def _tc_body(z_ref, w_ref, b_ref, o_ref):
    acc = jnp.dot(z_ref[:, :D], w_ref[...],
                  preferred_element_type=jnp.float32)
    o_ref[...] = jnp.tanh(acc + b_ref[...])


def _tc_decode_first(z, W_dec, b2):
    return pl.pallas_call(
        _tc_body,
        grid=(STEPS,),
        in_specs=[
            pl.BlockSpec((ROWS, 2 * D), lambda i: (i, 0)),
            pl.BlockSpec((D, HOP), lambda i: (0, 0)),
            pl.BlockSpec((1, HOP), lambda i: (0, 0)),
        ],
        out_specs=pl.BlockSpec((ROWS, HOP), lambda i: (i, 0)),
        out_shape=jax.ShapeDtypeStruct((F, HOP), jnp.float32),
    )(z, W_dec, b2)


def _tc_decode_chunk(z, W_dec, b2, prev, h):
    def body(z_ref, w_ref, b_ref, p_ref, o_ref):
        _tc_body(z_ref, w_ref, b_ref, o_ref)

    return pl.pallas_call(
        body,
        grid=(STEPS,),
        in_specs=[
            pl.BlockSpec((ROWS, 2 * D), lambda i: (i, 0)),
            pl.BlockSpec((D, HOP), lambda i: (0, 0)),
            pl.BlockSpec((1, HOP), lambda i: (0, 0)),
            pl.BlockSpec(memory_space=pl.ANY),
        ],
        out_specs=pl.BlockSpec((ROWS, HOP), lambda i: (h * STEPS + i, 0)),
        out_shape=jax.ShapeDtypeStruct((F, HOP), jnp.float32),
        input_output_aliases={3: 0},
    )(z, W_dec, b2, prev)


def kernel(ids, cb1, cb2, cb3, W_dec, b_dec):
    ids_flat = ids.reshape(-1).astype(jnp.int32)
    b2 = b_dec.reshape(1, HOP)
    zs = [_sc_gather_combine(ids_flat, cb1, cb2, cb3, h) for h in range(CH)]
    out = _tc_decode_first(zs[0], W_dec, b2)
    for h in range(1, CH):
        out = _tc_decode_chunk(zs[h], W_dec, b2, out, h)
    return out.reshape(B, 1, 4 * T * HOP)


# speedup vs baseline: 1.9473x; 1.0726x over previous
import functools

import jax
import jax.numpy as jnp
from jax import lax
from jax.experimental import pallas as pl
from jax.experimental.pallas import tpu as pltpu
from jax.experimental.pallas import tpu_sc as plsc

B = 16
T = 1024
K = 4096
D = 64
HOP = 128
C = B * T
F = 4 * C
CH = 1

NC = 2
NS = 16
NW = NC * NS
C_CH = C // CH
F_CH = 4 * C_CH
C_W = C_CH // NW
F_W = 4 * C_W
NSUB = 8
C_SUB = C_W // NSUB
F_SUB = 4 * C_SUB
H_SUB = F_SUB // 2
SEG = 128
NSEG = F_SUB // SEG
NSEG_W = F_W // SEG
NSEG2 = F_W // 2 // SEG


def _sc_gather_combine(ids_flat, cb1, cb2, cb3, h):
    mesh = plsc.VectorSubcoreMesh(core_axis_name="c", subcore_axis_name="s")

    @functools.partial(
        pl.kernel,
        out_type=jax.ShapeDtypeStruct((F_CH, 2 * D), jnp.float32),
        mesh=mesh,
        scratch_types=[
            pltpu.VMEM((C_W * 7,), jnp.int32),
            pltpu.VMEM((NSEG_W, SEG), jnp.int32),
            pltpu.VMEM((NSEG2, SEG), jnp.int32),
            pltpu.VMEM((NSUB, C_SUB), jnp.int32),
            pltpu.VMEM((2, F_SUB, D), jnp.float32),
            pltpu.VMEM((2, H_SUB, D), jnp.float32),
            pltpu.VMEM((2, C_SUB, D), jnp.float32),
            pltpu.SemaphoreType.DMA,
            pltpu.SemaphoreType.DMA,
            pltpu.SemaphoreType.DMA,
            pltpu.SemaphoreType.DMA,
            pltpu.SemaphoreType.DMA,
        ],
        compiler_params=pltpu.CompilerParams(needs_layout_passes=False,
                                             use_tc_tiling_on_sc=False),
    )
    def k(ids_hbm, cb1_hbm, cb2_hbm, cb3_hbm, z_hbm,
          ids_v, idx3_v, idx2_v, idx1_v, z_v, l2_v, l1_v,
          sem_ids, sem_g0, sem_g1, sem_aux, sem_out):
        wid = lax.axis_index("s") * NC + lax.axis_index("c")
        base_c = h * C_CH + wid * C_W
        pltpu.async_copy(ids_hbm.at[pl.ds(base_c * 7, C_W * 7)],
                         ids_v, sem_ids).wait()
        lane = lax.broadcasted_iota(jnp.int32, (16,), 0)

        def build3(i, carry):
            f = lane + i * 16
            s = f & 3
            idx3_v[i >> 3, pl.ds((i & 7) * 16, 16)] = plsc.load_gather(
                ids_v, [(f >> 2) * 7 + (3 + s)]) - 2 * K
            return carry

        def build2(i, carry):
            hh = lane + i * 16
            idx2_v[i >> 3, pl.ds((i & 7) * 16, 16)] = plsc.load_gather(
                ids_v, [(hh >> 1) * 7 + (1 + (hh & 1))]) - K
            return carry

        def build1(i, carry):
            t = lane + i * 16
            idx1_v[i >> 2, pl.ds((i & 3) * 16, 16)] = plsc.load_gather(
                ids_v, [t * 7])
            return carry

        lax.fori_loop(0, F_W // 16, build3, 0)
        lax.fori_loop(0, F_W // 2 // 16, build2, 0)
        lax.fori_loop(0, F_W // 4 // 16, build1, 0)

        sem_g = (sem_g0, sem_g1)

        def fire(sub, buf):
            ds_ = [pltpu.async_copy(
                       cb3_hbm.at[idx3_v.at[NSEG * sub + g]],
                       z_v.at[buf, pl.ds(g * SEG, SEG)],
                       sem_g[buf])
                   for g in range(NSEG)]
            ds_.append(pltpu.async_copy(
                cb2_hbm.at[idx2_v.at[sub]], l2_v.at[buf], sem_aux))
            ds_.append(pltpu.async_copy(
                cb1_hbm.at[idx1_v.at[sub]], l1_v.at[buf], sem_aux))
            return ds_

        def fire_out(sub, buf):
            base_f = wid * F_W + sub * F_SUB
            return pltpu.async_copy(
                z_v.at[buf],
                z_hbm.at[pl.ds(base_f, F_SUB), pl.ds(0, D)], sem_out)

        def add_phase(buf):
            def body(tc, carry):
                a = [l1_v[buf, tc, pl.ds(c * 16, 16)] for c in range(4)]
                for u in range(2):
                    acc = [a[c] + l2_v[buf, 2 * tc + u, pl.ds(c * 16, 16)]
                           for c in range(4)]
                    for s2 in range(2):
                        fr = 4 * tc + 2 * u + s2
                        for c in range(4):
                            plsc.addupdate(
                                z_v.at[buf, fr, pl.ds(c * 16, 16)], acc[c])
                return carry
            lax.fori_loop(0, C_SUB, body, 0)

        gat_d = [None] * NSUB
        out_d = [None] * NSUB
        gat_d[0] = fire(0, 0)
        for sub in range(NSUB):
            buf = sub % 2
            for dsc in gat_d[sub]:
                dsc.wait()
            if sub + 1 < NSUB:
                if sub >= 1:
                    out_d[sub - 1].wait()
                gat_d[sub + 1] = fire(sub + 1, 1 - buf)
            add_phase(buf)
            out_d[sub] = fire_out(sub, buf)
        out_d[NSUB - 2].wait()
        out_d[NSUB - 1].wait()

    return k(ids_flat, cb1, cb2, cb3)


ROWS = 16384
STEPS = F_CH // ROWS


def _tc_body(z_ref, w_ref, b_ref, o_ref):
    acc = jnp.dot(z_ref[:, :D], w_ref[...],
                  preferred_element_type=jnp.float32)
    o_ref[...] = jnp.tanh(acc + b_ref[...])


def _tc_decode_first(z, W_dec, b2):
    return pl.pallas_call(
        _tc_body,
        grid=(STEPS,),
        in_specs=[
            pl.BlockSpec((ROWS, 2 * D), lambda i: (i, 0)),
            pl.BlockSpec((D, HOP), lambda i: (0, 0)),
            pl.BlockSpec((1, HOP), lambda i: (0, 0)),
        ],
        out_specs=pl.BlockSpec((ROWS, HOP), lambda i: (i, 0)),
        out_shape=jax.ShapeDtypeStruct((F, HOP), jnp.float32),
    )(z, W_dec, b2)


def _tc_decode_chunk(z, W_dec, b2, prev, h):
    def body(z_ref, w_ref, b_ref, p_ref, o_ref):
        _tc_body(z_ref, w_ref, b_ref, o_ref)

    return pl.pallas_call(
        body,
        grid=(STEPS,),
        in_specs=[
            pl.BlockSpec((ROWS, 2 * D), lambda i: (i, 0)),
            pl.BlockSpec((D, HOP), lambda i: (0, 0)),
            pl.BlockSpec((1, HOP), lambda i: (0, 0)),
            pl.BlockSpec(memory_space=pl.ANY),
        ],
        out_specs=pl.BlockSpec((ROWS, HOP), lambda i: (h * STEPS + i, 0)),
        out_shape=jax.ShapeDtypeStruct((F, HOP), jnp.float32),
        input_output_aliases={3: 0},
    )(z, W_dec, b2, prev)


def kernel(ids, cb1, cb2, cb3, W_dec, b_dec):
    ids_flat = ids.reshape(-1).astype(jnp.int32)
    b2 = b_dec.reshape(1, HOP)
    zs = [_sc_gather_combine(ids_flat, cb1, cb2, cb3, h) for h in range(CH)]
    out = _tc_decode_first(zs[0], W_dec, b2)
    for h in range(1, CH):
        out = _tc_decode_chunk(zs[h], W_dec, b2, out, h)
    return out.reshape(B, 1, 4 * T * HOP)


# speedup vs baseline: 1.9492x; 1.0010x over previous
import functools

import jax
import jax.numpy as jnp
from jax import lax
from jax.experimental import pallas as pl
from jax.experimental.pallas import tpu as pltpu
from jax.experimental.pallas import tpu_sc as plsc

B = 16
T = 1024
K = 4096
D = 64
HOP = 128
C = B * T
F = 4 * C
CH = 1

NC = 2
NS = 16
NW = NC * NS
C_CH = C // CH
F_CH = 4 * C_CH
C_W = C_CH // NW
F_W = 4 * C_W
NSUB = 8
C_SUB = C_W // NSUB
F_SUB = 4 * C_SUB
H_SUB = F_SUB // 2
SEG = 128
NSEG = F_SUB // SEG
NSEG_W = F_W // SEG
NSEG2 = F_W // 2 // SEG


def _sc_gather_combine(ids_flat, cb1, cb2, cb3, h):
    mesh = plsc.VectorSubcoreMesh(core_axis_name="c", subcore_axis_name="s")

    @functools.partial(
        pl.kernel,
        out_type=jax.ShapeDtypeStruct((F_CH, 2 * D), jnp.float32),
        mesh=mesh,
        scratch_types=[
            pltpu.VMEM((C_W * 7,), jnp.int32),
            pltpu.VMEM((NSEG_W, SEG), jnp.int32),
            pltpu.VMEM((NSEG2, SEG), jnp.int32),
            pltpu.VMEM((NSUB, C_SUB), jnp.int32),
            pltpu.VMEM((2, F_SUB, D), jnp.float32),
            pltpu.VMEM((2, H_SUB, D), jnp.float32),
            pltpu.VMEM((2, C_SUB, D), jnp.float32),
            pltpu.SemaphoreType.DMA,
            pltpu.SemaphoreType.DMA,
            pltpu.SemaphoreType.DMA,
            pltpu.SemaphoreType.DMA,
            pltpu.SemaphoreType.DMA,
        ],
        compiler_params=pltpu.CompilerParams(needs_layout_passes=False,
                                             use_tc_tiling_on_sc=False),
    )
    def k(ids_hbm, cb1_hbm, cb2_hbm, cb3_hbm, z_hbm,
          ids_v, idx3_v, idx2_v, idx1_v, z_v, l2_v, l1_v,
          sem_ids, sem_g0, sem_g1, sem_aux, sem_out):
        wid = lax.axis_index("s") * NC + lax.axis_index("c")
        base_c = h * C_CH + wid * C_W
        pltpu.async_copy(ids_hbm.at[pl.ds(base_c * 7, C_W * 7)],
                         ids_v, sem_ids).wait()
        lane = lax.broadcasted_iota(jnp.int32, (16,), 0)

        def build3(i, carry):
            f = lane + i * 16
            s = f & 3
            idx3_v[i >> 3, pl.ds((i & 7) * 16, 16)] = plsc.load_gather(
                ids_v, [(f >> 2) * 7 + (3 + s)]) - 2 * K
            return carry

        def build2(i, carry):
            hh = lane + i * 16
            idx2_v[i >> 3, pl.ds((i & 7) * 16, 16)] = plsc.load_gather(
                ids_v, [(hh >> 1) * 7 + (1 + (hh & 1))]) - K
            return carry

        def build1(i, carry):
            t = lane + i * 16
            idx1_v[i >> 2, pl.ds((i & 3) * 16, 16)] = plsc.load_gather(
                ids_v, [t * 7])
            return carry

        lax.fori_loop(0, F_W // 16, build3, 0)
        lax.fori_loop(0, F_W // 2 // 16, build2, 0)
        lax.fori_loop(0, F_W // 4 // 16, build1, 0)

        sem_g = (sem_g0, sem_g1)

        def fire(sub, buf):
            ds_ = [pltpu.async_copy(
                       cb3_hbm.at[idx3_v.at[NSEG * sub + g]],
                       z_v.at[buf, pl.ds(g * SEG, SEG)],
                       sem_g[buf])
                   for g in range(NSEG)]
            ds_.append(pltpu.async_copy(
                cb2_hbm.at[idx2_v.at[sub]], l2_v.at[buf], sem_aux))
            ds_.append(pltpu.async_copy(
                cb1_hbm.at[idx1_v.at[sub]], l1_v.at[buf], sem_aux))
            return ds_

        def fire_out(sub, buf):
            base_f = wid * F_W + sub * F_SUB
            return pltpu.async_copy(
                z_v.at[buf],
                z_hbm.at[pl.ds(base_f, F_SUB), pl.ds(0, D)], sem_out)

        def add_phase(buf):
            def body(tc, carry):
                a = [l1_v[buf, tc, pl.ds(c * 16, 16)] for c in range(4)]
                for u in range(2):
                    acc = [a[c] + l2_v[buf, 2 * tc + u, pl.ds(c * 16, 16)]
                           for c in range(4)]
                    for s2 in range(2):
                        fr = 4 * tc + 2 * u + s2
                        for c in range(4):
                            plsc.addupdate(
                                z_v.at[buf, fr, pl.ds(c * 16, 16)], acc[c])
                return carry
            lax.fori_loop(0, C_SUB, body, 0)

        gat_d = [None] * NSUB
        out_d = [None] * NSUB
        gat_d[0] = fire(0, 0)
        for sub in range(NSUB):
            buf = sub % 2
            for dsc in gat_d[sub]:
                dsc.wait()
            if sub + 1 < NSUB:
                if sub >= 1:
                    out_d[sub - 1].wait()
                gat_d[sub + 1] = fire(sub + 1, 1 - buf)
            add_phase(buf)
            out_d[sub] = fire_out(sub, buf)
        out_d[NSUB - 2].wait()
        out_d[NSUB - 1].wait()

    return k(ids_flat, cb1, cb2, cb3)


ROWS = 16384
STEPS = F_CH // ROWS


def _tc_body(z_ref, w_ref, b_ref, o_ref):
    acc = jnp.dot(z_ref[:, :D], w_ref[...],
                  preferred_element_type=jnp.float32)
    o_ref[...] = jnp.tanh(acc + b_ref[...])


def _tc_decode(z, W_dec, b2):
    return pl.pallas_call(
        _tc_body,
        grid=(STEPS,),
        in_specs=[
            pl.BlockSpec((ROWS, 2 * D), lambda i: (i, 0)),
            pl.BlockSpec((D, HOP), lambda i: (0, 0)),
            pl.BlockSpec((1, HOP), lambda i: (0, 0)),
        ],
        out_specs=pl.BlockSpec((ROWS, HOP), lambda i: (i, 0)),
        out_shape=jax.ShapeDtypeStruct((F, HOP), jnp.float32),
    )(z, W_dec, b2)


def kernel(ids, cb1, cb2, cb3, W_dec, b_dec):
    ids_flat = ids.reshape(-1).astype(jnp.int32)
    b2 = b_dec.reshape(1, HOP)
    z = _sc_gather_combine(ids_flat, cb1, cb2, cb3, 0)
    out = _tc_decode(z, W_dec, b2)
    return out.reshape(B, 1, 4 * T * HOP)
